# Initial kernel scaffold; baseline (speedup 1.0000x reference)
import jax
import jax.numpy as jnp
from jax.experimental import pallas as pl


def kernel(input_embeds, edge_index, input_index, W1, b1, W2, b2):
    raise NotImplementedError("write your pallas kernel here")


# speedup vs baseline: 4.2846x; 4.2846x over previous
import functools

import jax
import jax.numpy as jnp
from jax import lax
from jax.experimental import pallas as pl
from jax.experimental.pallas import tpu as pltpu
from jax.experimental.pallas import tpu_sc as plsc

N = 10000
E2 = 640000
D = 128
NR = 64
RANGE = 160
NPAD = NR * RANGE
CAP = 12448
SCAN_CHUNK = 2560
N_SCAN = E2 // SCAN_CHUNK
GCH = 128
NBLK = NPAD // 256

_mesh = plsc.VectorSubcoreMesh(core_axis_name="c", subcore_axis_name="s")


def _wid():
    return lax.axis_index("s") * 2 + lax.axis_index("c")


def _k1_body(src_hbm, dst_hbm, cnt_o, srcl_o, locl_o, degrep_o,
             dstc, srcc, sl0, ll0, sl1, ll1, hist, degb, cntv):
    wid = _wid()
    r0 = wid * 2
    lo0 = r0 * RANGE
    lo1 = lo0 + RANGE
    iota = lax.iota(jnp.int32, 16)

    def chunk_body(c, carry):
        pltpu.sync_copy(dst_hbm.at[pl.ds(c * SCAN_CHUNK, SCAN_CHUNK)], dstc)
        pltpu.sync_copy(src_hbm.at[pl.ds(c * SCAN_CHUNK, SCAN_CHUNK)], srcc)

        def vec_body(j, cnts):
            cnt0, cnt1 = cnts
            d = dstc[pl.ds(j * 16, 16)]
            s = srcc[pl.ds(j * 16, 16)]

            def one(lo, list_s, list_l, cnt):
                m = (d >= lo) & (d < lo + RANGE)
                cs = plsc.cumsum(m.astype(jnp.int32))
                pos = cnt + cs - 1
                plsc.store_scatter(list_s, [pos], s, mask=m)
                plsc.store_scatter(list_l, [pos], d - lo, mask=m)
                return cnt + cs[15]

            cnt0 = one(lo0, sl0, ll0, cnt0)
            cnt1 = one(lo1, sl1, ll1, cnt1)
            return (cnt0, cnt1)

        return lax.fori_loop(0, SCAN_CHUNK // 16, vec_body, carry)

    cnt0, cnt1 = lax.fori_loop(0, N_SCAN, chunk_body,
                               (jnp.int32(0), jnp.int32(0)))

    for r_i, (list_s, list_l, cnt) in enumerate(((sl0, ll0, cnt0),
                                                 (sl1, ll1, cnt1))):
        r = r0 + r_i
        pad_l = jnp.full((16,), RANGE, jnp.int32)
        pad_s = jnp.zeros((16,), jnp.int32)
        for i in range(8):
            pos = cnt + iota + (16 * i)
            plsc.store_scatter(list_l, [pos], pad_l)
            plsc.store_scatter(list_s, [pos], pad_s)
        cnt_pad = ((cnt + 127) // 128) * 128
        cntv[...] = jnp.full((16,), cnt_pad, jnp.int32)
        pltpu.sync_copy(cntv, cnt_o.at[pl.ds(r * 16, 16)])
        pltpu.sync_copy(list_s, srcl_o.at[pl.ds(r * CAP, CAP)])
        pltpu.sync_copy(list_l, locl_o.at[pl.ds(r * CAP, CAP)])

        def hz(i, _):
            hist[i] = 0.0
            return 0
        lax.fori_loop(0, RANGE + 16, hz, 0)

        def he(g, _):
            vl = list_l[pl.ds(16 * g, 16)]
            for k in range(16):
                v = vl[k]
                hist[v] = hist[v] + 1.0
            return 0
        lax.fori_loop(0, cnt_pad // 16, he, 0)

        def db(i, _):
            spl = jnp.full((16,), hist[i], jnp.float32)
            for l in range(8):
                degb[i, pl.ds(16 * l, 16)] = spl
            return 0
        lax.fori_loop(0, RANGE, db, 0)
        pltpu.sync_copy(degb, degrep_o.at[pl.ds(r * RANGE, RANGE)])


_k1 = pl.kernel(
    _k1_body,
    out_type=(jax.ShapeDtypeStruct((NR * 16,), jnp.int32),
              jax.ShapeDtypeStruct((NR * CAP,), jnp.int32),
              jax.ShapeDtypeStruct((NR * CAP,), jnp.int32),
              jax.ShapeDtypeStruct((NPAD, D), jnp.float32)),
    mesh=_mesh,
    compiler_params=pltpu.CompilerParams(needs_layout_passes=False),
    scratch_types=[
        pltpu.VMEM((SCAN_CHUNK,), jnp.int32),
        pltpu.VMEM((SCAN_CHUNK,), jnp.int32),
        pltpu.VMEM((CAP,), jnp.int32),
        pltpu.VMEM((CAP,), jnp.int32),
        pltpu.VMEM((CAP,), jnp.int32),
        pltpu.VMEM((CAP,), jnp.int32),
        pltpu.SMEM((RANGE + 16,), jnp.float32),
        pltpu.VMEM((RANGE, D), jnp.float32),
        pltpu.VMEM((16,), jnp.int32),
    ],
)


def _k2_body(h_hbm, srcl, locl, cnt_hbm, sum_o, ssq_o, mx_o, mn_o,
             sacc, qacc, xacc, nacc, idxv, locv, rows, cntv, sem):
    wid = _wid()
    zero = jnp.zeros((16,), jnp.float32)
    big = jnp.full((16,), 3.0e38, jnp.float32)

    for r_i in range(2):
        r = wid * 2 + r_i

        def init(i, _):
            for l in range(8):
                sl_ = pl.ds(16 * l, 16)
                sacc[i, sl_] = zero
                qacc[i, sl_] = zero
                xacc[i, sl_] = -big
                nacc[i, sl_] = big
            return 0
        lax.fori_loop(0, RANGE + 1, init, 0)

        pltpu.sync_copy(cnt_hbm.at[pl.ds(r * 16, 16)], cntv)
        nch = cntv[...][0] // GCH

        def chunk(c, _):
            pltpu.sync_copy(srcl.at[pl.ds(r * CAP + c * GCH, GCH)], idxv)
            pltpu.sync_copy(locl.at[pl.ds(r * CAP + c * GCH, GCH)], locv)
            pltpu.async_copy(h_hbm.at[idxv], rows, sem).wait()

            def group(g, _):
                vl = locv[pl.ds(16 * g, 16)]
                for k in range(16):
                    v = vl[k]
                    e = g * 16 + k
                    for l in range(8):
                        sl_ = pl.ds(16 * l, 16)
                        x = rows[e, sl_]
                        sacc[v, sl_] = sacc[v, sl_] + x
                        qacc[v, sl_] = qacc[v, sl_] + x * x
                        xacc[v, sl_] = jnp.maximum(xacc[v, sl_], x)
                        nacc[v, sl_] = jnp.minimum(nacc[v, sl_], x)
                return 0
            return lax.fori_loop(0, GCH // 16, group, 0)
        lax.fori_loop(0, nch, chunk, 0)

        row_sl = pl.ds(r * RANGE, RANGE)
        pltpu.sync_copy(sacc.at[pl.ds(0, RANGE)], sum_o.at[row_sl])
        pltpu.sync_copy(qacc.at[pl.ds(0, RANGE)], ssq_o.at[row_sl])
        pltpu.sync_copy(xacc.at[pl.ds(0, RANGE)], mx_o.at[row_sl])
        pltpu.sync_copy(nacc.at[pl.ds(0, RANGE)], mn_o.at[row_sl])


_stat = jax.ShapeDtypeStruct((NPAD, D), jnp.float32)
_k2 = pl.kernel(
    _k2_body,
    out_type=(_stat, _stat, _stat, _stat),
    mesh=_mesh,
    compiler_params=pltpu.CompilerParams(needs_layout_passes=False),
    scratch_types=[
        pltpu.VMEM((RANGE + 1, D), jnp.float32),
        pltpu.VMEM((RANGE + 1, D), jnp.float32),
        pltpu.VMEM((RANGE + 1, D), jnp.float32),
        pltpu.VMEM((RANGE + 1, D), jnp.float32),
        pltpu.VMEM((GCH,), jnp.int32),
        pltpu.VMEM((GCH,), jnp.int32),
        pltpu.VMEM((GCH, D), jnp.float32),
        pltpu.VMEM((16,), jnp.int32),
        pltpu.SemaphoreType.DMA,
    ],
)


def _k3a_body(deg_ref, rdeg_ref, amp_ref, att_ref, g_ref, acc_ref):
    p = pl.program_id(0)
    deg = deg_ref[...]
    logdeg = jnp.log(deg + 1.0)

    @pl.when(jnp.logical_and(p == 0, pl.program_id(1) == 0))
    def _():
        acc_ref[0, 0] = 0.0

    @pl.when(p == 0)
    def _():
        acc_ref[0, 0] = acc_ref[0, 0] + jnp.sum(logdeg)

    @pl.when(p == 1)
    def _():
        delta = acc_ref[0, 0] / (float(N) * float(D))
        rdeg_ref[...] = 1.0 / jnp.maximum(deg, 1.0)
        amp_ref[...] = logdeg / delta
        att_ref[...] = jnp.where(logdeg > 0, delta / jnp.maximum(logdeg, 1e-5), 1.0)
        g_ref[...] = jnp.where(deg > 0, 1.0, 0.0)


_blk = pl.BlockSpec((256, D), lambda p, j: (j, 0))
_k3a = pl.pallas_call(
    _k3a_body,
    grid=(2, NBLK),
    in_specs=[_blk],
    out_specs=[_blk, _blk, _blk, _blk],
    out_shape=(_stat, _stat, _stat, _stat),
    scratch_shapes=[pltpu.SMEM((1, 1), jnp.float32)],
)


def _k3_body(h_ref, s_ref, q_ref, x_ref, n_ref, rdeg_ref, amp_ref, att_ref,
             g_ref, w_ref, b_ref, o_ref):
    rdeg = rdeg_ref[...]
    mean = s_ref[...] * rdeg
    msq = q_ref[...] * rdeg
    var = jnp.maximum(msq - mean * mean, 0.0)
    std = jnp.sqrt(var + 1e-5)
    g = g_ref[...]
    mx = jnp.where(g > 0, x_ref[...], 0.0)
    mn = jnp.where(g > 0, n_ref[...], 0.0)
    amp = amp_ref[...]
    att = att_ref[...]
    feats = (h_ref[...], mean, mx, mn, std,
             mean * amp, mx * amp, mn * amp, std * amp,
             mean * att, mx * att, mn * att, std * att)
    acc = jnp.broadcast_to(b_ref[...], (256, D))
    for i in range(13):
        acc = acc + jnp.dot(feats[i], w_ref[i],
                            preferred_element_type=jnp.float32)
    o_ref[...] = jnp.maximum(acc, 0.0)


_blk1 = pl.BlockSpec((256, D), lambda j: (j, 0))
_k3 = pl.pallas_call(
    _k3_body,
    grid=(NBLK,),
    in_specs=[_blk1] * 9 + [pl.BlockSpec((13, D, D), lambda j: (0, 0, 0)),
                            pl.BlockSpec((1, D), lambda j: (0, 0))],
    out_specs=_blk1,
    out_shape=jax.ShapeDtypeStruct((NPAD, D), jnp.float32),
)


def kernel(input_embeds, edge_index, input_index, W1, b1, W2, b2):
    del input_index
    e0 = edge_index[0]
    e1 = edge_index[1]
    src = jnp.concatenate([e0, e1])
    dst = jnp.concatenate([e1, e0])
    h = jnp.concatenate(
        [input_embeds,
         jnp.zeros((NPAD - input_embeds.shape[0], D), jnp.float32)], axis=0)

    cnts, srcl, locl, degrep = _k1(src, dst)
    rdeg, amp, att, g = _k3a(degrep)

    for W, b in ((W1, b1), (W2, b2)):
        Wr = W.reshape(13, D, D)
        br = b.reshape(1, D)
        s, q, mx, mn = _k2(h, srcl, locl, cnts)
        h = _k3(h, s, q, mx, mn, rdeg, amp, att, g, Wr, br)
    return h[:N]
